# baseline (speedup 1.0000x reference)
import jax
import jax.numpy as jnp
from jax import lax
from jax.experimental import pallas as pl
from jax.experimental.pallas import tpu as pltpu
from jax.experimental.pallas import tpu_sc as plsc

K = 100000
D = 128
L = 16
NW = 16
NPW = 6256
NPW_LAST = K - (NW - 1) * NPW
NV = NPW // L
NV_LAST = NPW_LAST // L
NACC = 4
UN = 8
NB = (NV - NACC) // UN
NB_LAST = (NV_LAST - NACC) // UN


def _lexmax(v1, i1, v2, i2):
    upd = (v2 > v1) | ((v2 == v1) & (i2 < i1))
    return jnp.where(upd, v2, v1), jnp.where(upd, i2, i1)


def _lexmin(v1, i1, v2, i2):
    upd = (v2 < v1) | ((v2 == v1) & (i2 < i1))
    return jnp.where(upd, v2, v1), jnp.where(upd, i2, i1)


def _sc_body(x_hbm, refs_hbm, posc_hbm, pcorl_hbm, negc_hbm, ncorl_hbm,
             stage_hbm, x_v, cand_v, all_v, idx_v, valsv_v, rows_v,
             sem, gsem):
    wid = lax.axis_index("s")
    base = wid * NPW
    last = wid == NW - 1

    @pl.when(jnp.logical_not(last))
    def _():
        pltpu.sync_copy(x_hbm.at[pl.ds(base, NPW)], x_v.at[pl.ds(0, NPW)])

    @pl.when(last)
    def _():
        pltpu.sync_copy(x_hbm.at[pl.ds(base, NPW_LAST)],
                        x_v.at[pl.ds(0, NPW_LAST)])

    lane = lax.iota(jnp.int32, L)
    flane = lane.astype(jnp.float32)
    fbase = base.astype(jnp.float32) + flane

    def step(j, jf, acc):
        maxv, maxj, minv, minj = acc
        v = x_v[pl.ds(j * L, L)]
        gt = v > maxv
        maxv = jnp.where(gt, v, maxv)
        maxj = jnp.where(gt, jf, maxj)
        lt = v < minv
        minv = jnp.where(lt, v, minv)
        minj = jnp.where(lt, jf, minj)
        return maxv, maxj, minv, minj

    accs = []
    for a in range(NACC):
        va = x_v[pl.ds(a * L, L)]
        ja = jnp.full((L,), float(a), jnp.float32)
        accs.append((va, ja, va, ja))

    def block(b, carry):
        accs = [tuple(carry[4 * a + t] for t in range(4)) for a in range(NACC)]
        j0 = NACC + b * UN
        jf0 = j0.astype(jnp.float32)
        for u in range(UN):
            a = u % NACC
            accs[a] = step(j0 + u, jf0 + float(u), accs[a])
        return tuple(x for acc in accs for x in acc)

    init = tuple(x for acc in accs for x in acc)
    nb = jnp.where(last, NB_LAST, NB)
    carry = lax.fori_loop(0, nb, block, init)

    def tail_step(j, carry):
        acc0 = tuple(carry[:4])
        acc0 = step(j, j.astype(jnp.float32), acc0)
        return acc0 + tuple(carry[4:])

    nv = jnp.where(last, NV_LAST, NV)
    carry = lax.fori_loop(NACC + UN * nb, nv, tail_step, carry)
    accs = [tuple(carry[4 * a + t] for t in range(4)) for a in range(NACC)]

    sixteen = jnp.float32(16.0)
    mxv, mxi = accs[0][0], accs[0][1] * sixteen + fbase
    mnv, mni = accs[0][2], accs[0][3] * sixteen + fbase
    for a in range(1, NACC):
        av, aj, iv, ij = accs[a]
        mxv, mxi = _lexmax(mxv, mxi, av, aj * sixteen + fbase)
        mnv, mni = _lexmin(mnv, mni, iv, ij * sixteen + fbase)

    cand_v[0, :] = mxv
    cand_v[1, :] = mxi
    cand_v[2, :] = mnv
    cand_v[3, :] = mni
    pltpu.sync_copy(cand_v, stage_hbm.at[wid])
    plsc.subcore_barrier()

    @pl.when(wid == 0)
    def _():
        pltpu.sync_copy(stage_hbm, all_v)
        gmaxv = all_v[0, 0, :]
        gmaxi = all_v[0, 1, :]
        gminv = all_v[0, 2, :]
        gmini = all_v[0, 3, :]
        for w in range(1, NW):
            gmaxv, gmaxi = _lexmax(gmaxv, gmaxi, all_v[w, 0, :], all_v[w, 1, :])
            gminv, gmini = _lexmin(gminv, gmini, all_v[w, 2, :], all_v[w, 3, :])

        for sh in (8, 4, 2, 1):
            perm = (lane + sh) & (L - 1)
            gmaxv, gmaxi = _lexmax(gmaxv, gmaxi, gmaxv[perm], gmaxi[perm])
            gminv, gmini = _lexmin(gminv, gmini, gminv[perm], gmini[perm])

        valsv_v[...] = jnp.where(lane == 0, gmaxv,
                                 jnp.where(lane == 8, gminv, 0.0))
        idx_v[...] = jnp.where(lane == 0, gmaxi.astype(jnp.int32),
                               jnp.where(lane == 1, gmini.astype(jnp.int32),
                                         0))
        c1 = pltpu.async_copy(valsv_v.at[pl.ds(0, 1)], pcorl_hbm, sem)
        c2 = pltpu.async_copy(valsv_v.at[pl.ds(8, 1)], ncorl_hbm, sem)
        g = pltpu.async_copy(refs_hbm.at[idx_v], rows_v, gsem)
        g.wait()
        c3 = pltpu.async_copy(rows_v.at[0], posc_hbm, sem)
        c4 = pltpu.async_copy(rows_v.at[1], negc_hbm, sem)
        c1.wait()
        c2.wait()
        c3.wait()
        c4.wait()


@jax.jit
def _infer(x, refs):
    mesh = plsc.VectorSubcoreMesh(
        core_axis_name="c", subcore_axis_name="s",
        num_cores=1, num_subcores=NW)
    f = pl.kernel(
        _sc_body,
        out_type=(
            jax.ShapeDtypeStruct((D,), jnp.float32),
            jax.ShapeDtypeStruct((1,), jnp.float32),
            jax.ShapeDtypeStruct((D,), jnp.float32),
            jax.ShapeDtypeStruct((1,), jnp.float32),
            jax.ShapeDtypeStruct((NW, 4, L), jnp.float32),
        ),
        mesh=mesh,
        scratch_types=[
            pltpu.VMEM((NPW,), jnp.float32),
            pltpu.VMEM((4, L), jnp.float32),
            pltpu.VMEM((NW, 4, L), jnp.float32),
            pltpu.VMEM((L,), jnp.int32),
            pltpu.VMEM((L,), jnp.float32),
            pltpu.VMEM((L, D), jnp.float32),
            pltpu.SemaphoreType.DMA,
            pltpu.SemaphoreType.DMA,
        ],
    )
    return f(x, refs)


def kernel(inputs, refs):
    posc, pcorl, negc, ncorl, _ = _infer(inputs, refs)
    return posc, jnp.reshape(pcorl, ()), negc, jnp.reshape(ncorl, ())
